# Initial kernel scaffold; baseline (speedup 1.0000x reference)
import jax
import jax.numpy as jnp
from jax.experimental import pallas as pl


def kernel(user_emb, item_emb, aug_user_emb, aug_item_emb, adj_values, adj_indices):
    raise NotImplementedError("write your pallas kernel here")


# speedup vs baseline: 2.5461x; 2.5461x over previous
import functools

import jax
import jax.numpy as jnp
from jax import lax
from jax.experimental import pallas as pl
from jax.experimental.pallas import tpu as pltpu
from jax.experimental.pallas import tpu_sc as plsc

NC = 2
NS = 16
LANES = 16
SUP = 1024
SUB = 128


@functools.lru_cache(maxsize=None)
def _build(N_PAD, NNZ_PAD, NG, interpret=False):
    NGC = NG // NC
    Q = NNZ_PAD // NS
    M = Q // SUP
    NSUB = SUP // SUB
    RT = N_PAD // NS
    CH = 448
    NCHK = RT // CH

    out_sds = jax.ShapeDtypeStruct((NG, N_PAD, LANES), jnp.float32)
    mesh = plsc.VectorSubcoreMesh(core_axis_name="c", subcore_axis_name="s",
                                  num_cores=NC, num_subcores=NS)

    @functools.partial(
        pl.kernel,
        out_type=[out_sds, out_sds, out_sds, out_sds],
        mesh=mesh,
        interpret=interpret,
        compiler_params=pltpu.CompilerParams(use_tc_tiling_on_sc=False),
        scratch_types=[
            pltpu.VMEM((SUP,), jnp.int32),
            pltpu.VMEM((NSUB, SUB), jnp.int32),
            pltpu.VMEM((SUP,), jnp.float32),
            pltpu.VMEM((SUB, LANES), jnp.float32),
            pltpu.VMEM((CH, LANES), jnp.float32),
            pltpu.VMEM((CH, LANES), jnp.float32),
            pltpu.VMEM_SHARED((N_PAD, LANES), jnp.float32),
        ],
    )
    def sacl_kernel(x3, cols, rows2d, vals, mean_out, y0, y1, y2,
                    col_v, row_v, val_v, gat_v, stage_v, stage2_v, acc):
        c = lax.axis_index("c")
        s = lax.axis_index("s")

        def zero_acc():
            def zb(e, carry):
                stage_v[e, :] = jnp.zeros((LANES,), jnp.float32)
                return carry
            lax.fori_loop(0, CH, zb, 0)

            def cp(k, carry):
                pltpu.sync_copy(stage_v, acc.at[pl.ds(s * RT + k * CH, CH)])
                return carry
            lax.fori_loop(0, NCHK, cp, 0)

        def scatter_pass(src, g):
            def sup_body(m, carry):
                base = s * Q + m * SUP
                pltpu.sync_copy(cols.at[pl.ds(base, SUP)], col_v)
                pltpu.sync_copy(vals.at[pl.ds(base, SUP)], val_v)
                pltpu.sync_copy(rows2d.at[pl.ds(s * (Q // SUB) + m * NSUB, NSUB)],
                                row_v)

                def sub_body(j, carry2):
                    pltpu.sync_copy(src.at[g].at[col_v.at[pl.ds(j * SUB, SUB)]],
                                    gat_v)

                    @plsc.parallel_loop(0, SUB // LANES, unroll=2)
                    def _scale(q):
                        vals16 = val_v[pl.ds(j * SUB + q * LANES, LANES)]
                        base = q * LANES
                        for e16 in range(LANES):
                            gat_v[base + e16, :] = (gat_v[base + e16, :]
                                                    * vals16[e16])

                    pltpu.sync_copy(gat_v, acc.at[row_v.at[j]], add=True)
                    return carry2
                lax.fori_loop(0, NSUB, sub_body, 0)
                return carry
            lax.fori_loop(0, M, sup_body, 0)

        def writeback(dst, g):
            def cp(k, carry):
                b = s * RT + k * CH
                pltpu.sync_copy(acc.at[pl.ds(b, CH)], stage_v)
                pltpu.sync_copy(stage_v, dst.at[g].at[pl.ds(b, CH)])
                return carry
            lax.fori_loop(0, NCHK, cp, 0)

        def mean_pass(g):
            third = jnp.float32(1.0 / 3.0)

            def cp(k, carry):
                b = s * RT + k * CH
                pltpu.sync_copy(y0.at[g].at[pl.ds(b, CH)], stage_v)
                pltpu.sync_copy(y1.at[g].at[pl.ds(b, CH)], stage2_v)

                @plsc.parallel_loop(0, CH, unroll=8)
                def _add1(e):
                    stage_v[e, :] = stage_v[e, :] + stage2_v[e, :]

                pltpu.sync_copy(y2.at[g].at[pl.ds(b, CH)], stage2_v)

                @plsc.parallel_loop(0, CH, unroll=8)
                def _add2(e):
                    stage_v[e, :] = (stage_v[e, :] + stage2_v[e, :]) * third

                pltpu.sync_copy(stage_v, mean_out.at[g].at[pl.ds(b, CH)])
                return carry
            lax.fori_loop(0, NCHK, cp, 0)

        def gi_body(gi, carry):
            g = c * NGC + gi
            for src, dst in ((x3, y0), (y0, y1), (y1, y2)):
                zero_acc()
                plsc.subcore_barrier()
                scatter_pass(src, g)
                plsc.subcore_barrier()
                writeback(dst, g)
                plsc.subcore_barrier()
            mean_pass(g)
            return carry
        lax.fori_loop(0, NGC, gi_body, 0)

    return sacl_kernel


def _run(user_emb, item_emb, aug_user_emb, aug_item_emb, adj_values,
         adj_indices, interpret=False):
    U, E = user_emb.shape
    I = item_emb.shape[0]
    N = U + I
    F = 2 * E
    NG = F // LANES

    RT = -(-N // (NS * 448)) * 448
    N_PAD = NS * RT

    ego = jnp.concatenate(
        [jnp.concatenate([user_emb, item_emb], axis=0),
         jnp.concatenate([aug_user_emb, aug_item_emb], axis=0)], axis=1)
    ego = jnp.pad(ego, ((0, N_PAD - N), (0, 0)))
    x3 = ego.reshape(N_PAD, NG, LANES).transpose(1, 0, 2)

    NNZ = adj_values.shape[0]
    Q = -(-NNZ // (NS * SUP)) * SUP
    NNZ_PAD = NS * Q
    pad = NNZ_PAD - NNZ
    rows_p = jnp.pad(adj_indices[0], (0, pad))
    cols_p = jnp.pad(adj_indices[1], (0, pad))
    vals_p = jnp.pad(adj_values, (0, pad))
    rows2d = rows_p.reshape(NNZ_PAD // SUB, SUB)

    mean3, _, _, _ = _build(N_PAD, NNZ_PAD, NG, interpret)(
        x3, cols_p, rows2d, vals_p)
    mean = mean3.transpose(1, 0, 2).reshape(N_PAD, F)[:N]
    rec, aug = mean[:, :E], mean[:, E:]
    return (rec[:U], rec[U:], aug[:U], aug[U:])


def kernel(user_emb, item_emb, aug_user_emb, aug_item_emb, adj_values,
           adj_indices):
    return _run(user_emb, item_emb, aug_user_emb, aug_item_emb, adj_values,
                adj_indices)


# speedup vs baseline: 4.7920x; 1.8821x over previous
import functools

import jax
import jax.numpy as jnp
from jax import lax
from jax.experimental import pallas as pl
from jax.experimental.pallas import tpu as pltpu
from jax.experimental.pallas import tpu_sc as plsc

NC = 2
NS = 16
LANES = 16
SUP = 1024
SUB = 128


@functools.lru_cache(maxsize=None)
def _build(N_PAD, NNZ_PAD, NG, interpret=False):
    NGC = NG // NC
    Q = NNZ_PAD // NS
    M = Q // SUP
    NSUB = SUP // SUB
    RT = N_PAD // NS
    CH = 448
    NCHK = RT // CH

    out_sds = jax.ShapeDtypeStruct((NG, N_PAD, LANES), jnp.float32)
    mesh = plsc.VectorSubcoreMesh(core_axis_name="c", subcore_axis_name="s",
                                  num_cores=NC, num_subcores=NS)

    @functools.partial(
        pl.kernel,
        out_type=[out_sds, out_sds, out_sds, out_sds],
        mesh=mesh,
        interpret=interpret,
        compiler_params=pltpu.CompilerParams(use_tc_tiling_on_sc=False),
        scratch_types=[
            pltpu.VMEM((2, SUP), jnp.int32),
            pltpu.VMEM((2, NSUB, SUB), jnp.int32),
            pltpu.VMEM((2, SUP), jnp.float32),
            pltpu.VMEM((2, SUB, LANES), jnp.float32),
            pltpu.VMEM((2, SUB, LANES), jnp.float32),
            pltpu.VMEM((CH, LANES), jnp.float32),
            pltpu.VMEM((CH, LANES), jnp.float32),
            pltpu.VMEM_SHARED((N_PAD, LANES), jnp.float32),
            pltpu.SemaphoreType.DMA((2,)),
            pltpu.SemaphoreType.DMA((2,)),
            pltpu.SemaphoreType.DMA((2,)),
        ],
    )
    def sacl_kernel(x3, cols, rows2d, vals, mean_out, y0, y1, y2,
                    col_v, row_v, val_v, gat_v, sct_v, stage_v, stage2_v, acc,
                    g_sem, s_sem, i_sem):
        c = lax.axis_index("c")
        s = lax.axis_index("s")

        def zero_acc():
            def zb(e, carry):
                stage_v[e, :] = jnp.zeros((LANES,), jnp.float32)
                return carry
            lax.fori_loop(0, CH, zb, 0)

            def cp(k, carry):
                pltpu.sync_copy(stage_v, acc.at[pl.ds(s * RT + k * CH, CH)])
                return carry
            lax.fori_loop(0, NCHK, cp, 0)

        NSUBT = Q // SUB

        def scatter_pass(src, g):
            base_t = s * Q

            def idx_issue(m2, b):
                sb = base_t + m2 * SUP
                pltpu.async_copy(cols.at[pl.ds(sb, SUP)], col_v.at[b],
                                 i_sem.at[b])
                pltpu.async_copy(rows2d.at[pl.ds(sb // SUB, NSUB)],
                                 row_v.at[b], i_sem.at[b])
                pltpu.async_copy(vals.at[pl.ds(sb, SUP)], val_v.at[b],
                                 i_sem.at[b])

            def idx_wait(b):
                pltpu.make_async_copy(cols.at[pl.ds(0, SUP)], col_v.at[b],
                                      i_sem.at[b]).wait()
                pltpu.make_async_copy(rows2d.at[pl.ds(0, NSUB)], row_v.at[b],
                                      i_sem.at[b]).wait()
                pltpu.make_async_copy(vals.at[pl.ds(0, SUP)], val_v.at[b],
                                      i_sem.at[b]).wait()

            def gather_issue(jj):
                p = jj % 2
                mb = (jj // NSUB) % 2
                off = (jj % NSUB) * SUB
                pltpu.async_copy(
                    src.at[g].at[col_v.at[mb].at[pl.ds(off, SUB)]],
                    gat_v.at[p], g_sem.at[p])

            def gather_wait(p):
                pltpu.make_async_copy(
                    src.at[g].at[col_v.at[0].at[pl.ds(0, SUB)]],
                    gat_v.at[p], g_sem.at[p]).wait()

            def scatter_issue(jj):
                p = jj % 2
                mb = (jj // NSUB) % 2
                kk = jj % NSUB
                pltpu.async_copy(sct_v.at[p], acc.at[row_v.at[mb].at[kk]],
                                 s_sem.at[p], add=True)

            def scatter_wait(p):
                pltpu.make_async_copy(sct_v.at[p], acc.at[row_v.at[0].at[0]],
                                      s_sem.at[p]).wait()

            idx_issue(0, 0)
            idx_wait(0)
            idx_issue(1, 1)
            gather_issue(0)
            gather_issue(1)

            def it(jj, carry):
                kk = lax.rem(jj, NSUB)
                m = lax.div(jj, NSUB)
                p = lax.rem(jj, 2)
                mb = lax.rem(m, 2)

                @pl.when(jnp.logical_and(kk == 2,
                                         jnp.logical_and(m >= 1, m < M - 1)))
                def _():
                    idx_issue(m + 1, 1 - mb)

                @pl.when(jnp.logical_and(kk == 6, m < M - 1))
                def _():
                    idx_wait(1 - mb)

                gather_wait(p)

                @pl.when(jj >= 2)
                def _():
                    scatter_wait(p)

                @plsc.parallel_loop(0, SUB // LANES, unroll=2)
                def _scale(q):
                    vals16 = val_v[mb, pl.ds(kk * SUB + q * LANES, LANES)]
                    qb = q * LANES
                    for e16 in range(LANES):
                        sct_v[p, qb + e16, :] = (gat_v[p, qb + e16, :]
                                                 * vals16[e16])

                scatter_issue(jj)

                @pl.when(jj + 2 < NSUBT)
                def _():
                    gather_issue(jj + 2)
                return carry
            lax.fori_loop(0, NSUBT, it, 0)
            scatter_wait(0)
            scatter_wait(1)

        def writeback(dst, g):
            def cp(k, carry):
                b = s * RT + k * CH
                pltpu.sync_copy(acc.at[pl.ds(b, CH)], stage_v)
                pltpu.sync_copy(stage_v, dst.at[g].at[pl.ds(b, CH)])
                return carry
            lax.fori_loop(0, NCHK, cp, 0)

        def mean_pass(g):
            third = jnp.float32(1.0 / 3.0)

            def cp(k, carry):
                b = s * RT + k * CH
                pltpu.sync_copy(y0.at[g].at[pl.ds(b, CH)], stage_v)
                pltpu.sync_copy(y1.at[g].at[pl.ds(b, CH)], stage2_v)

                @plsc.parallel_loop(0, CH, unroll=8)
                def _add1(e):
                    stage_v[e, :] = stage_v[e, :] + stage2_v[e, :]

                pltpu.sync_copy(y2.at[g].at[pl.ds(b, CH)], stage2_v)

                @plsc.parallel_loop(0, CH, unroll=8)
                def _add2(e):
                    stage_v[e, :] = (stage_v[e, :] + stage2_v[e, :]) * third

                pltpu.sync_copy(stage_v, mean_out.at[g].at[pl.ds(b, CH)])
                return carry
            lax.fori_loop(0, NCHK, cp, 0)

        def gi_body(gi, carry):
            g = c * NGC + gi
            for src, dst in ((x3, y0), (y0, y1), (y1, y2)):
                zero_acc()
                plsc.subcore_barrier()
                scatter_pass(src, g)
                plsc.subcore_barrier()
                writeback(dst, g)
                plsc.subcore_barrier()
            mean_pass(g)
            return carry
        lax.fori_loop(0, NGC, gi_body, 0)

    return sacl_kernel


def _run(user_emb, item_emb, aug_user_emb, aug_item_emb, adj_values,
         adj_indices, interpret=False):
    U, E = user_emb.shape
    I = item_emb.shape[0]
    N = U + I
    F = 2 * E
    NG = F // LANES

    RT = -(-N // (NS * 448)) * 448
    N_PAD = NS * RT

    ego = jnp.concatenate(
        [jnp.concatenate([user_emb, item_emb], axis=0),
         jnp.concatenate([aug_user_emb, aug_item_emb], axis=0)], axis=1)
    ego = jnp.pad(ego, ((0, N_PAD - N), (0, 0)))
    x3 = ego.reshape(N_PAD, NG, LANES).transpose(1, 0, 2)

    NNZ = adj_values.shape[0]
    Q = -(-NNZ // (NS * SUP)) * SUP
    NNZ_PAD = NS * Q
    pad = NNZ_PAD - NNZ
    rows_p = jnp.pad(adj_indices[0], (0, pad))
    cols_p = jnp.pad(adj_indices[1], (0, pad))
    vals_p = jnp.pad(adj_values, (0, pad))
    rows2d = rows_p.reshape(NNZ_PAD // SUB, SUB)

    mean3, _, _, _ = _build(N_PAD, NNZ_PAD, NG, interpret)(
        x3, cols_p, rows2d, vals_p)
    mean = mean3.transpose(1, 0, 2).reshape(N_PAD, F)[:N]
    rec, aug = mean[:, :E], mean[:, E:]
    return (rec[:U], rec[U:], aug[:U], aug[U:])


def kernel(user_emb, item_emb, aug_user_emb, aug_item_emb, adj_values,
           adj_indices):
    return _run(user_emb, item_emb, aug_user_emb, aug_item_emb, adj_values,
                adj_indices)


# speedup vs baseline: 6.1747x; 1.2885x over previous
import functools

import jax
import jax.numpy as jnp
from jax import lax
from jax.experimental import pallas as pl
from jax.experimental.pallas import tpu as pltpu
from jax.experimental.pallas import tpu_sc as plsc

NC = 2
NS = 16
LANES = 16
SUP = 512
SUB = 128


@functools.lru_cache(maxsize=None)
def _build(N_PAD, NNZ_PAD, NG, interpret=False):
    NGC = NG // NC
    Q = NNZ_PAD // NS
    M = Q // SUP
    NSUB = SUP // SUB
    RT = N_PAD // NS
    CH = 224
    NCHK = RT // CH

    out_sds = jax.ShapeDtypeStruct((NG, N_PAD, LANES), jnp.float32)
    mesh = plsc.VectorSubcoreMesh(core_axis_name="c", subcore_axis_name="s",
                                  num_cores=NC, num_subcores=NS)

    @functools.partial(
        pl.kernel,
        out_type=[out_sds, out_sds, out_sds, out_sds],
        mesh=mesh,
        interpret=interpret,
        compiler_params=pltpu.CompilerParams(use_tc_tiling_on_sc=False),
        scratch_types=[
            pltpu.VMEM((3, SUP), jnp.int32),
            pltpu.VMEM((4, NSUB, SUB), jnp.int32),
            pltpu.VMEM((3, SUP), jnp.float32),
            pltpu.VMEM((2, 2 * SUB, LANES), jnp.float32),
            pltpu.VMEM((2, 2 * SUB, LANES), jnp.float32),
            pltpu.VMEM((CH, LANES), jnp.float32),
            pltpu.VMEM((CH, LANES), jnp.float32),
            pltpu.VMEM_SHARED((N_PAD, LANES), jnp.float32),
            pltpu.SemaphoreType.DMA((2,)),
            pltpu.SemaphoreType.DMA((2,)),
            pltpu.SemaphoreType.DMA((3,)),
            pltpu.SemaphoreType.DMA((4,)),
        ],
    )
    def sacl_kernel(x3, cols, rows2d, vals, mean_out, y0, y1, y2,
                    col_v, row_v, val_v, gat_v, sct_v, stage_v, stage2_v, acc,
                    g_sem, s_sem, i_sem, r_sem):
        c = lax.axis_index("c")
        s = lax.axis_index("s")

        def zero_acc():
            def zb(e, carry):
                stage_v[e, :] = jnp.zeros((LANES,), jnp.float32)
                return carry
            lax.fori_loop(0, CH, zb, 0)

            def cp(k, carry):
                pltpu.sync_copy(stage_v, acc.at[pl.ds(s * RT + k * CH, CH)])
                return carry
            lax.fori_loop(0, NCHK, cp, 0)

        SUBI = 2 * SUB
        NIT = Q // SUBI
        IPS = SUP // SUBI

        def scatter_pass(src, g):
            base_t = s * Q

            def idx_issue(m2):
                b3 = m2 % 3
                b4 = m2 % 4
                sb = base_t + m2 * SUP
                pltpu.async_copy(cols.at[pl.ds(sb, SUP)], col_v.at[b3],
                                 i_sem.at[b3])
                pltpu.async_copy(vals.at[pl.ds(sb, SUP)], val_v.at[b3],
                                 i_sem.at[b3])
                pltpu.async_copy(rows2d.at[pl.ds(sb // SUB, NSUB)],
                                 row_v.at[b4], r_sem.at[b4])

            def idx_wait(m2):
                b3 = m2 % 3
                b4 = m2 % 4
                pltpu.make_async_copy(cols.at[pl.ds(0, SUP)], col_v.at[b3],
                                      i_sem.at[b3]).wait()
                pltpu.make_async_copy(vals.at[pl.ds(0, SUP)], val_v.at[b3],
                                      i_sem.at[b3]).wait()
                pltpu.make_async_copy(rows2d.at[pl.ds(0, NSUB)], row_v.at[b4],
                                      r_sem.at[b4]).wait()

            def gather_issue(jj):
                p = jj % 2
                b3 = (jj // IPS) % 3
                off = (jj % IPS) * SUBI
                for h in range(2):
                    pltpu.async_copy(
                        src.at[g].at[col_v.at[b3].at[pl.ds(off + h * SUB,
                                                           SUB)]],
                        gat_v.at[p].at[pl.ds(h * SUB, SUB)], g_sem.at[p])

            def gather_wait(p):
                for h in range(2):
                    pltpu.make_async_copy(
                        src.at[g].at[col_v.at[0].at[pl.ds(0, SUB)]],
                        gat_v.at[p].at[pl.ds(h * SUB, SUB)],
                        g_sem.at[p]).wait()

            def scatter_issue(jj):
                p = jj % 2
                b4 = (jj // IPS) % 4
                kk = jj % IPS
                for h in range(2):
                    pltpu.async_copy(
                        sct_v.at[p].at[pl.ds(h * SUB, SUB)],
                        acc.at[row_v.at[b4].at[2 * kk + h]],
                        s_sem.at[p], add=True)

            def scatter_wait(p):
                for h in range(2):
                    pltpu.make_async_copy(
                        sct_v.at[p].at[pl.ds(h * SUB, SUB)],
                        acc.at[row_v.at[0].at[0]], s_sem.at[p]).wait()

            idx_issue(0)
            idx_issue(1)
            idx_wait(0)
            idx_wait(1)
            gather_issue(0)
            gather_issue(1)

            def it(jj, carry):
                kk = lax.rem(jj, IPS)
                m = lax.div(jj, IPS)
                p = lax.rem(jj, 2)
                b3 = lax.rem(m, 3)

                gather_wait(p)

                @pl.when(jj >= 2)
                def _():
                    scatter_wait(p)

                @pl.when(jnp.logical_and(kk == 0, m + 2 < M))
                def _():
                    idx_issue(m + 2)

                @pl.when(jnp.logical_and(kk == 0,
                                         jnp.logical_and(m >= 1, m + 1 < M)))
                def _():
                    idx_wait(m + 1)

                @plsc.parallel_loop(0, SUBI // LANES, unroll=4)
                def _scale(q):
                    vals16 = val_v[b3, pl.ds(kk * SUBI + q * LANES, LANES)]
                    qb = q * LANES
                    for e16 in range(LANES):
                        sct_v[p, qb + e16, :] = (gat_v[p, qb + e16, :]
                                                 * vals16[e16])

                scatter_issue(jj)

                @pl.when(jj + 2 < NIT)
                def _():
                    gather_issue(jj + 2)
                return carry
            lax.fori_loop(0, NIT, it, 0)
            scatter_wait(0)
            scatter_wait(1)

        def writeback(dst, g):
            def cp(k, carry):
                b = s * RT + k * CH
                pltpu.sync_copy(acc.at[pl.ds(b, CH)], stage_v)
                pltpu.sync_copy(stage_v, dst.at[g].at[pl.ds(b, CH)])
                return carry
            lax.fori_loop(0, NCHK, cp, 0)

        def mean_pass(g):
            third = jnp.float32(1.0 / 3.0)

            def cp(k, carry):
                b = s * RT + k * CH
                pltpu.sync_copy(y0.at[g].at[pl.ds(b, CH)], stage_v)
                pltpu.sync_copy(y1.at[g].at[pl.ds(b, CH)], stage2_v)

                @plsc.parallel_loop(0, CH, unroll=8)
                def _add1(e):
                    stage_v[e, :] = stage_v[e, :] + stage2_v[e, :]

                pltpu.sync_copy(y2.at[g].at[pl.ds(b, CH)], stage2_v)

                @plsc.parallel_loop(0, CH, unroll=8)
                def _add2(e):
                    stage_v[e, :] = (stage_v[e, :] + stage2_v[e, :]) * third

                pltpu.sync_copy(stage_v, mean_out.at[g].at[pl.ds(b, CH)])
                return carry
            lax.fori_loop(0, NCHK, cp, 0)

        def gi_body(gi, carry):
            g = c * NGC + gi
            for src, dst in ((x3, y0), (y0, y1), (y1, y2)):
                zero_acc()
                plsc.subcore_barrier()
                scatter_pass(src, g)
                plsc.subcore_barrier()
                writeback(dst, g)
                plsc.subcore_barrier()
            mean_pass(g)
            return carry
        lax.fori_loop(0, NGC, gi_body, 0)

    return sacl_kernel


def _run(user_emb, item_emb, aug_user_emb, aug_item_emb, adj_values,
         adj_indices, interpret=False):
    U, E = user_emb.shape
    I = item_emb.shape[0]
    N = U + I
    F = 2 * E
    NG = F // LANES

    RT = -(-N // (NS * 448)) * 448
    N_PAD = NS * RT

    ego = jnp.concatenate(
        [jnp.concatenate([user_emb, item_emb], axis=0),
         jnp.concatenate([aug_user_emb, aug_item_emb], axis=0)], axis=1)
    ego = jnp.pad(ego, ((0, N_PAD - N), (0, 0)))
    x3 = ego.reshape(N_PAD, NG, LANES).transpose(1, 0, 2)

    NNZ = adj_values.shape[0]
    Q = -(-NNZ // (NS * SUP)) * SUP
    NNZ_PAD = NS * Q
    pad = NNZ_PAD - NNZ
    rows_p = jnp.pad(adj_indices[0], (0, pad))
    cols_p = jnp.pad(adj_indices[1], (0, pad))
    vals_p = jnp.pad(adj_values, (0, pad))
    rows2d = rows_p.reshape(NNZ_PAD // SUB, SUB)

    mean3, _, _, _ = _build(N_PAD, NNZ_PAD, NG, interpret)(
        x3, cols_p, rows2d, vals_p)
    mean = mean3.transpose(1, 0, 2).reshape(N_PAD, F)[:N]
    rec, aug = mean[:, :E], mean[:, E:]
    return (rec[:U], rec[U:], aug[:U], aug[U:])


def kernel(user_emb, item_emb, aug_user_emb, aug_item_emb, adj_values,
           adj_indices):
    return _run(user_emb, item_emb, aug_user_emb, aug_item_emb, adj_values,
                adj_indices)
